# baseline (speedup 1.0000x reference)
import functools

import jax
import jax.numpy as jnp
from jax import lax
from jax.experimental import pallas as pl
from jax.experimental.pallas import tpu as pltpu
from jax.experimental.pallas import tpu_sc as plsc

VOC_DIM = 64
BATCH = 16384
N_LOOKUPS = 4
CHUNK = 128


def _make_sc_gather(V, D, B_total):
    info = plsc.get_sparse_core_info()
    NC, NS = info.num_cores, info.num_subcores
    NW = NC * NS
    b_per_w = B_total // NW
    n_chunks = b_per_w // CHUNK
    mesh = plsc.VectorSubcoreMesh(core_axis_name="c", subcore_axis_name="s")

    @functools.partial(
        pl.kernel,
        mesh=mesh,
        out_type=jax.ShapeDtypeStruct((B_total, D), jnp.float32),
        scratch_types=[
            pltpu.VMEM((b_per_w,), jnp.int32),
            pltpu.VMEM((2, CHUNK, D), jnp.float32),
            pltpu.SemaphoreType.DMA,
            pltpu.SemaphoreType.DMA,
        ],
        compiler_params=pltpu.CompilerParams(use_tc_tiling_on_sc=False),
    )
    def gather_k(idx_hbm, table_hbm, out_hbm, idx_v, rows_v, sem0, sem1):
        wid = lax.axis_index("s") * NC + lax.axis_index("c")
        base = wid * b_per_w
        pltpu.sync_copy(idx_hbm.at[pl.ds(base, b_per_w)], idx_v)
        sems = (sem0, sem1)
        copies = [None, None]
        for j in range(n_chunks + 1):
            if j < n_chunks:
                b = j & 1
                copies[b] = pltpu.async_copy(
                    table_hbm.at[idx_v.at[pl.ds(j * CHUNK, CHUNK)]],
                    rows_v.at[b],
                    sems[b],
                )
            if j >= 1:
                b2 = (j - 1) & 1
                copies[b2].wait()
                pltpu.sync_copy(
                    rows_v.at[b2],
                    out_hbm.at[pl.ds(base + (j - 1) * CHUNK, CHUNK)],
                )

    return gather_k


_sc_gather = None


def _get_sc_gather():
    global _sc_gather
    if _sc_gather is None:
        _sc_gather = _make_sc_gather(None, VOC_DIM, N_LOOKUPS * BATCH)
    return _sc_gather


def _proj_body(g_ref, wt_ref, o_ref):
    acc = jnp.dot(g_ref[0], wt_ref[0], preferred_element_type=jnp.float32)
    for k in range(1, N_LOOKUPS):
        acc += jnp.dot(g_ref[k], wt_ref[k], preferred_element_type=jnp.float32)
    o_ref[...] = acc


def _tc_project(gathered, wt_stack):
    BB = 1024
    grid = (BATCH // BB,)
    return pl.pallas_call(
        _proj_body,
        grid=grid,
        in_specs=[
            pl.BlockSpec((N_LOOKUPS, BB, VOC_DIM), lambda i: (0, i, 0)),
            pl.BlockSpec((N_LOOKUPS, VOC_DIM, VOC_DIM), lambda i: (0, 0, 0)),
        ],
        out_specs=pl.BlockSpec((BB, VOC_DIM), lambda i: (i, 0)),
        out_shape=jax.ShapeDtypeStruct((BATCH, VOC_DIM), jnp.float32),
    )(gathered, wt_stack)


def kernel(x1, x2, x4, x5, codebook, W1, W2, W3, W4):
    idx_all = jnp.concatenate([x1, x2, x4, x5]).astype(jnp.int32)
    gathered = _get_sc_gather()(idx_all, codebook)
    gathered = gathered.reshape(N_LOOKUPS, BATCH, VOC_DIM)
    wt_stack = jnp.stack([W1.T, W2.T, W3.T, W4.T])
    return _tc_project(gathered, wt_stack)
